# baseline (speedup 1.0000x reference)
import functools

import jax
import jax.numpy as jnp
from jax import lax
from jax.experimental import pallas as pl
from jax.experimental.pallas import tpu as pltpu
from jax.experimental.pallas import tpu_sc as plsc

SEQ = 8192
BATCH = 4
D = 768

S_SC = 3072
S_TC = SEQ - S_SC

NUM_CORES = 2
NUM_SUBCORES = 16
NW = NUM_CORES * NUM_SUBCORES
ROWS_PER_W = S_SC // NW
NBUF = 2
CHUNK = 48
NCHUNK = ROWS_PER_W // CHUNK


def _sc_body(pe_hbm, out_hbm, *scratch):
    bufs = scratch[:NBUF]
    rsems = scratch[NBUF:2 * NBUF]
    wsems = scratch[2 * NBUF:3 * NBUF]
    wid = lax.axis_index("s") * NUM_CORES + lax.axis_index("c")
    base = wid * ROWS_PER_W

    reads = [None] * NCHUNK
    writes = [None] * NCHUNK

    def start_read(k):
        j = k % NBUF
        reads[k] = pltpu.async_copy(
            pe_hbm.at[pl.ds(base + k * CHUNK, CHUNK)], bufs[j], rsems[j])

    start_read(0)
    for i in range(NCHUNK):
        j = i % NBUF
        if i + 1 < NCHUNK:
            if i + 1 - NBUF >= 0:
                for c in writes[i + 1 - NBUF]:
                    c.wait()
            start_read(i + 1)
        reads[i].wait()
        row0 = base + i * CHUNK
        writes[i] = [
            pltpu.async_copy(bufs[j], out_hbm.at[pl.ds(row0, CHUNK), b], wsems[j])
            for b in range(BATCH)
        ]
    for k in range(max(0, NCHUNK - NBUF), NCHUNK):
        for c in writes[k]:
            c.wait()


_sc_bcast = functools.partial(
    pl.kernel,
    out_type=jax.ShapeDtypeStruct((S_SC, BATCH, D), jnp.float32),
    mesh=plsc.VectorSubcoreMesh(
        core_axis_name="c", subcore_axis_name="s",
        num_cores=NUM_CORES, num_subcores=NUM_SUBCORES),
    scratch_types=(
        [pltpu.VMEM((CHUNK, D), jnp.float32) for _ in range(NBUF)]
        + [pltpu.SemaphoreType.DMA for _ in range(2 * NBUF)]
    ),
)(_sc_body)


RB = 64


def _tc_body(pe_ref, out_ref):
    x = pe_ref[...]
    y = x.reshape(RB, 6, 128)
    for b in range(BATCH):
        out_ref[:, pl.ds(b * 6, 6), :] = y


def _tc_bcast(pe):
    return pl.pallas_call(
        _tc_body,
        grid=(S_TC // RB,),
        in_specs=[pl.BlockSpec((RB, D), lambda i: (i + S_SC // RB, 0))],
        out_specs=pl.BlockSpec((RB, BATCH * 6, 128), lambda i: (i, 0, 0)),
        out_shape=jax.ShapeDtypeStruct((S_TC, BATCH * 6, 128), jnp.float32),
    )(pe)


@jax.jit
def kernel(z, pos_embed):
    del z
    a = _sc_bcast(pos_embed)
    b = _tc_bcast(pos_embed)
    flat = jnp.concatenate(
        [a.reshape(S_SC * BATCH * D), b.reshape(S_TC * BATCH * D)])
    return flat.reshape(SEQ, BATCH, D)


# speedup vs baseline: 4.8990x; 4.8990x over previous
import functools

import jax
import jax.numpy as jnp
from jax import lax
from jax.experimental import pallas as pl
from jax.experimental.pallas import tpu as pltpu
from jax.experimental.pallas import tpu_sc as plsc

SEQ = 8192
BATCH = 4
D = 768

NUM_CORES = 2
NUM_SUBCORES = 16
NW = NUM_CORES * NUM_SUBCORES
ROWS_PER_W = SEQ // NW
NBUF = 2
CHUNK = 64
NCHUNK = ROWS_PER_W // CHUNK


def _body(pe_hbm, out_hbm, *scratch):
    bufs = scratch[:NBUF]
    rsems = scratch[NBUF:2 * NBUF]
    wsems = scratch[2 * NBUF:3 * NBUF]
    wid = lax.axis_index("s") * NUM_CORES + lax.axis_index("c")
    base = wid * ROWS_PER_W

    reads = [None] * NCHUNK
    writes = [None] * NCHUNK

    def start_read(k):
        j = k % NBUF
        reads[k] = pltpu.async_copy(
            pe_hbm.at[pl.ds(base + k * CHUNK, CHUNK)], bufs[j], rsems[j])

    start_read(0)
    for i in range(NCHUNK):
        j = i % NBUF
        if i + 1 < NCHUNK:
            if i + 1 - NBUF >= 0:
                for c in writes[i + 1 - NBUF]:
                    c.wait()
            start_read(i + 1)
        reads[i].wait()
        row0 = base + i * CHUNK
        writes[i] = [
            pltpu.async_copy(bufs[j], out_hbm.at[pl.ds(row0, CHUNK), b], wsems[j])
            for b in range(BATCH)
        ]
    for k in range(max(0, NCHUNK - NBUF), NCHUNK):
        for c in writes[k]:
            c.wait()


_bcast = functools.partial(
    pl.kernel,
    out_type=jax.ShapeDtypeStruct((SEQ, BATCH, D), jnp.float32),
    mesh=plsc.VectorSubcoreMesh(
        core_axis_name="c", subcore_axis_name="s",
        num_cores=NUM_CORES, num_subcores=NUM_SUBCORES),
    scratch_types=(
        [pltpu.VMEM((CHUNK, D), jnp.float32) for _ in range(NBUF)]
        + [pltpu.SemaphoreType.DMA for _ in range(2 * NBUF)]
    ),
)(_body)


@jax.jit
def kernel(z, pos_embed):
    del z
    return _bcast(pos_embed)


# speedup vs baseline: 4.9567x; 1.0118x over previous
import functools

import jax
import jax.numpy as jnp
from jax import lax
from jax.experimental import pallas as pl
from jax.experimental.pallas import tpu as pltpu
from jax.experimental.pallas import tpu_sc as plsc

SEQ = 8192
BATCH = 4
D = 768

NUM_CORES = 2
NUM_SUBCORES = 16
NW = NUM_CORES * NUM_SUBCORES
ROWS_PER_W = SEQ // NW
NBUF = 2
CHUNK = 64
NCHUNK = ROWS_PER_W // CHUNK


def _body(pe_hbm, out_hbm, buf, rsem, wsem):
    wid = lax.axis_index("s") * NUM_CORES + lax.axis_index("c")
    base = wid * ROWS_PER_W

    def chunk_slot(k):
        return pl.ds((k % NBUF) * CHUNK, CHUNK)

    def start_read(k):
        pltpu.async_copy(
            pe_hbm.at[pl.ds(base + k * CHUNK, CHUNK)], buf.at[chunk_slot(k)],
            rsem)

    def drain_writes_of_one_chunk():
        for b in range(BATCH):
            pltpu.make_async_copy(
                buf.at[pl.ds(0, CHUNK)],
                out_hbm.at[pl.ds(base, CHUNK), b], wsem).wait()

    start_read(0)

    def step(i, carry):
        pltpu.make_async_copy(
            pe_hbm.at[pl.ds(base, CHUNK)], buf.at[pl.ds(0, CHUNK)], rsem).wait()

        @pl.when(i + 1 < NCHUNK)
        def _():
            @pl.when(i >= 1)
            def _():
                drain_writes_of_one_chunk()
            start_read(i + 1)

        row0 = base + i * CHUNK
        for b in range(BATCH):
            pltpu.async_copy(
                buf.at[chunk_slot(i)], out_hbm.at[pl.ds(row0, CHUNK), b], wsem)
        return carry

    lax.fori_loop(0, NCHUNK, step, 0)
    drain_writes_of_one_chunk()
    drain_writes_of_one_chunk()


_bcast = functools.partial(
    pl.kernel,
    out_type=jax.ShapeDtypeStruct((SEQ, BATCH, D), jnp.float32),
    mesh=plsc.VectorSubcoreMesh(
        core_axis_name="c", subcore_axis_name="s",
        num_cores=NUM_CORES, num_subcores=NUM_SUBCORES),
    scratch_types=[
        pltpu.VMEM((NBUF * CHUNK, D), jnp.float32),
        pltpu.SemaphoreType.DMA,
        pltpu.SemaphoreType.DMA,
    ],
)(_body)


@jax.jit
def kernel(z, pos_embed):
    del z
    return _bcast(pos_embed)


# speedup vs baseline: 4.9709x; 1.0029x over previous
import functools

import jax
import jax.numpy as jnp
from jax import lax
from jax.experimental import pallas as pl
from jax.experimental.pallas import tpu as pltpu
from jax.experimental.pallas import tpu_sc as plsc

SEQ = 8192
BATCH = 4
D = 768

NUM_CORES = 2
NUM_SUBCORES = 16
NW = NUM_CORES * NUM_SUBCORES
ROWS_PER_W = SEQ // NW
NBUF = 2
CHUNK = 64
NCHUNK = ROWS_PER_W // CHUNK


def _body(pe_hbm, out_hbm, buf, rsem, wsem):
    wid = lax.axis_index("c") * NUM_SUBCORES + lax.axis_index("s")
    base = wid * ROWS_PER_W

    def chunk_slot(k):
        return pl.ds((k % NBUF) * CHUNK, CHUNK)

    def start_read(k):
        pltpu.async_copy(
            pe_hbm.at[pl.ds(base + k * CHUNK, CHUNK)], buf.at[chunk_slot(k)],
            rsem)

    def drain_writes_of_one_chunk():
        for b in range(BATCH):
            pltpu.make_async_copy(
                buf.at[pl.ds(0, CHUNK)],
                out_hbm.at[pl.ds(base, CHUNK), b], wsem).wait()

    start_read(0)

    def step(i, carry):
        pltpu.make_async_copy(
            pe_hbm.at[pl.ds(base, CHUNK)], buf.at[pl.ds(0, CHUNK)], rsem).wait()

        @pl.when(i + 1 < NCHUNK)
        def _():
            @pl.when(i >= 1)
            def _():
                drain_writes_of_one_chunk()
            start_read(i + 1)

        row0 = base + i * CHUNK
        for b in range(BATCH):
            pltpu.async_copy(
                buf.at[chunk_slot(i)], out_hbm.at[pl.ds(row0, CHUNK), b], wsem)
        return carry

    lax.fori_loop(0, NCHUNK, step, 0)
    drain_writes_of_one_chunk()
    drain_writes_of_one_chunk()


_bcast = functools.partial(
    pl.kernel,
    out_type=jax.ShapeDtypeStruct((SEQ, BATCH, D), jnp.float32),
    mesh=plsc.VectorSubcoreMesh(
        core_axis_name="c", subcore_axis_name="s",
        num_cores=NUM_CORES, num_subcores=NUM_SUBCORES),
    scratch_types=[
        pltpu.VMEM((NBUF * CHUNK, D), jnp.float32),
        pltpu.SemaphoreType.DMA,
        pltpu.SemaphoreType.DMA,
    ],
)(_body)


@jax.jit
def kernel(z, pos_embed):
    del z
    return _bcast(pos_embed)
